# baseline (speedup 1.0000x reference)
import functools

import jax
import jax.numpy as jnp
from jax import lax
from jax.experimental import pallas as pl
from jax.experimental.pallas import tpu as pltpu
from jax.experimental.pallas import tpu_sc as plsc

NCOND = 1000
ED = 64
SPATIAL = 256
WPAD = 1024
L = 16
RCHUNK = 8


def _make_lookup(B: int):
    info = plsc.get_sparse_core_info()
    nc, ns = info.num_cores, info.num_subcores
    nw = nc * ns
    bpw = B // nw
    assert B % nw == 0 and bpw % (2 * RCHUNK) == 0
    mesh = plsc.VectorSubcoreMesh(core_axis_name="c", subcore_axis_name="s")

    @functools.partial(
        pl.kernel,
        mesh=mesh,
        compiler_params=pltpu.CompilerParams(
            needs_layout_passes=False, use_tc_tiling_on_sc=False),
        out_type=jax.ShapeDtypeStruct((B, ED, SPATIAL), jnp.float32),
        scratch_types=[
            pltpu.VMEM((ED * WPAD,), jnp.float32),
            pltpu.VMEM((bpw,), jnp.int32),
            pltpu.VMEM((ED,), jnp.float32),
            pltpu.VMEM((ED,), jnp.float32),
            pltpu.VMEM((ED,), jnp.float32),
            pltpu.VMEM((RCHUNK, ED, L), jnp.float32),
            pltpu.VMEM((RCHUNK, ED, L), jnp.float32),
            pltpu.SemaphoreType.DMA,
            pltpu.SemaphoreType.DMA,
        ],
    )
    def lookup(w_hbm, idx_hbm, b_hbm, out_hbm,
               w_v, idx_v, b_v, mh_v, emb_v, blk0, blk1, sem0, sem1):
        wid = lax.axis_index("s") * nc + lax.axis_index("c")
        base = wid * bpw

        pltpu.sync_copy(w_hbm, w_v)
        pltpu.sync_copy(idx_hbm.at[pl.ds(base, bpw)], idx_v)
        pltpu.sync_copy(b_hbm, b_v)

        cmax = idx_v[pl.ds(0, L)]
        for g in range(1, bpw // L):
            cmax = jnp.maximum(cmax, idx_v[pl.ds(L * g, L)])
        has_fg = cmax[0] >= NCOND
        for i in range(1, L):
            has_fg = has_fg | (cmax[i] >= NCOND)

        row_off = tuple(
            (lax.iota(jnp.int32, L) + L * g) * WPAD for g in range(ED // L))

        @pl.when(has_fg)
        def _():
            def jbody(j, accs):
                jj = jnp.full((L,), j, jnp.int32)
                return tuple(
                    accs[g] + plsc.load_gather(w_v, [row_off[g] + jj])
                    for g in range(ED // L))
            accs = lax.fori_loop(
                1, NCOND, jbody,
                tuple(jnp.zeros((L,), jnp.float32) for _ in range(ED // L)))
            for g in range(ED // L):
                mh_v[pl.ds(L * g, L)] = accs[g]

        def compute_chunk(q, blk):
            def rbody(rl, carry):
                r = q * RCHUNK + rl
                cc = plsc.load_gather(idx_v, [jnp.full((L,), r, jnp.int32)])
                normal = cc < NCOND
                for g in range(ED // L):
                    wval = plsc.load_gather(w_v, [row_off[g] + cc])
                    mhv = mh_v[pl.ds(L * g, L)]
                    bv = b_v[pl.ds(L * g, L)]
                    emb_v[pl.ds(L * g, L)] = jnp.where(normal, wval, mhv) + bv

                def ebody(e, c2):
                    vv = plsc.load_gather(
                        emb_v, [jnp.full((L,), e, jnp.int32)])
                    blk[rl, e, pl.ds(0, L)] = vv
                    return c2
                lax.fori_loop(0, ED, ebody, 0)
                return carry
            lax.fori_loop(0, RCHUNK, rbody, 0)

        nq = bpw // RCHUNK

        def fire(q, blk, sem):
            compute_chunk(q, blk)
            r0 = base + q * RCHUNK
            for k in range(SPATIAL // L):
                pltpu.make_async_copy(
                    blk, out_hbm.at[pl.ds(r0, RCHUNK), :, pl.ds(L * k, L)],
                    sem).start()

        def drain(q, blk, sem):
            r0 = base + q * RCHUNK
            for k in range(SPATIAL // L):
                pltpu.make_async_copy(
                    blk, out_hbm.at[pl.ds(r0, RCHUNK), :, pl.ds(L * k, L)],
                    sem).wait()

        def pbody(p, carry):
            @pl.when(p >= 1)
            def _():
                drain(2 * p - 2, blk0, sem0)
            fire(2 * p, blk0, sem0)

            @pl.when(p >= 1)
            def _():
                drain(2 * p - 1, blk1, sem1)
            fire(2 * p + 1, blk1, sem1)
            return carry

        lax.fori_loop(0, nq // 2, pbody, 0)
        drain(nq - 2, blk0, sem0)
        drain(nq - 1, blk1, sem1)

    return lookup


def kernel(condition, spatial_shape, W, b):
    dims = jnp.asarray(spatial_shape)
    one = (dims[0] - 4 + dims[1] - 8 + dims[2] - 8 + 1).astype(jnp.float32)
    B = condition.shape[0]
    w_pad = jnp.pad(W.astype(jnp.float32) * one, ((0, 0), (0, WPAD - NCOND)))
    b_eff = b.astype(jnp.float32) * one
    idx = condition.astype(jnp.int32)
    out = _make_lookup(B)(w_pad.reshape(ED * WPAD), idx, b_eff)
    return out.reshape(B, ED, 4, 8, 8)


# speedup vs baseline: 1.4082x; 1.4082x over previous
import functools

import jax
import jax.numpy as jnp
from jax import lax
from jax.experimental import pallas as pl
from jax.experimental.pallas import tpu as pltpu
from jax.experimental.pallas import tpu_sc as plsc

NCOND = 1000
ED = 64
SPATIAL = 256
HALF = 128
WPAD = 1024
L = 16


def _make_lookup(B: int):
    info = plsc.get_sparse_core_info()
    nc, ns = info.num_cores, info.num_subcores
    nw = nc * ns
    bpw = B // nw
    assert B % nw == 0 and bpw % 2 == 0
    mesh = plsc.VectorSubcoreMesh(core_axis_name="c", subcore_axis_name="s")

    @functools.partial(
        pl.kernel,
        mesh=mesh,
        compiler_params=pltpu.CompilerParams(needs_layout_passes=False),
        out_type=jax.ShapeDtypeStruct((B, ED, SPATIAL), jnp.float32),
        scratch_types=[
            pltpu.VMEM((ED * WPAD,), jnp.float32),
            pltpu.VMEM((bpw,), jnp.int32),
            pltpu.VMEM((ED,), jnp.float32),
            pltpu.VMEM((ED,), jnp.float32),
            pltpu.VMEM((ED,), jnp.float32),
            pltpu.VMEM((ED, HALF), jnp.float32),
            pltpu.VMEM((ED, HALF), jnp.float32),
            pltpu.SemaphoreType.DMA,
            pltpu.SemaphoreType.DMA,
        ],
    )
    def lookup(w_hbm, idx_hbm, b_hbm, out_hbm,
               w_v, idx_v, b_v, mh_v, emb_v, buf0, buf1, sem0, sem1):
        wid = lax.axis_index("s") * nc + lax.axis_index("c")
        base = wid * bpw

        pltpu.sync_copy(w_hbm, w_v)
        pltpu.sync_copy(idx_hbm.at[pl.ds(base, bpw)], idx_v)
        pltpu.sync_copy(b_hbm, b_v)

        cmax = idx_v[pl.ds(0, L)]
        for g in range(1, bpw // L):
            cmax = jnp.maximum(cmax, idx_v[pl.ds(L * g, L)])
        has_fg = cmax[0] >= NCOND
        for i in range(1, L):
            has_fg = has_fg | (cmax[i] >= NCOND)

        row_off = tuple(
            (lax.iota(jnp.int32, L) + L * g) * WPAD for g in range(ED // L))

        @pl.when(has_fg)
        def _():
            def jbody(j, accs):
                jj = jnp.full((L,), j, jnp.int32)
                return tuple(
                    accs[g] + plsc.load_gather(w_v, [row_off[g] + jj])
                    for g in range(ED // L))
            accs = lax.fori_loop(
                1, NCOND, jbody,
                tuple(jnp.zeros((L,), jnp.float32) for _ in range(ED // L)))
            for g in range(ED // L):
                mh_v[pl.ds(L * g, L)] = accs[g]

        def compute_row(r, buf):
            cc = plsc.load_gather(idx_v, [jnp.full((L,), r, jnp.int32)])
            normal = cc < NCOND
            for g in range(ED // L):
                wval = plsc.load_gather(w_v, [row_off[g] + cc])
                mhv = mh_v[pl.ds(L * g, L)]
                bv = b_v[pl.ds(L * g, L)]
                emb_v[pl.ds(L * g, L)] = jnp.where(normal, wval, mhv) + bv

            def ebody(e, c2):
                vv = plsc.load_gather(emb_v, [jnp.full((L,), e, jnp.int32)])
                for k in range(HALF // L):
                    buf[e, pl.ds(L * k, L)] = vv
                return c2
            lax.fori_loop(0, ED, ebody, 0)

        def fire(r, buf, sem):
            compute_row(r, buf)
            row = base + r
            pltpu.make_async_copy(
                buf, out_hbm.at[row, :, pl.ds(0, HALF)], sem).start()
            pltpu.make_async_copy(
                buf, out_hbm.at[row, :, pl.ds(HALF, HALF)], sem).start()

        def drain(r, buf, sem):
            row = base + r
            pltpu.make_async_copy(
                buf, out_hbm.at[row, :, pl.ds(0, HALF)], sem).wait()
            pltpu.make_async_copy(
                buf, out_hbm.at[row, :, pl.ds(HALF, HALF)], sem).wait()

        def qbody(q, carry):
            @pl.when(q >= 1)
            def _():
                drain(2 * q - 2, buf0, sem0)
            fire(2 * q, buf0, sem0)

            @pl.when(q >= 1)
            def _():
                drain(2 * q - 1, buf1, sem1)
            fire(2 * q + 1, buf1, sem1)
            return carry

        lax.fori_loop(0, bpw // 2, qbody, 0)
        drain(bpw - 2, buf0, sem0)
        drain(bpw - 1, buf1, sem1)

    return lookup


def kernel(condition, spatial_shape, W, b):
    dims = jnp.asarray(spatial_shape)
    one = (dims[0] - 4 + dims[1] - 8 + dims[2] - 8 + 1).astype(jnp.float32)
    B = condition.shape[0]
    w_pad = jnp.pad(W.astype(jnp.float32) * one, ((0, 0), (0, WPAD - NCOND)))
    b_eff = b.astype(jnp.float32) * one
    idx = condition.astype(jnp.int32)
    out = _make_lookup(B)(w_pad.reshape(ED * WPAD), idx, b_eff)
    return out.reshape(B, ED, 4, 8, 8)


# speedup vs baseline: 4.6603x; 3.3094x over previous
import functools

import jax
import jax.numpy as jnp
from jax import lax
from jax.experimental import pallas as pl
from jax.experimental.pallas import tpu as pltpu
from jax.experimental.pallas import tpu_sc as plsc

NCOND = 1000
ED = 64
WPAD = 1024
L = 16
D, WD, H = 4, 8, 8
BLANE = 128


def _make_lookup(B: int):
    info = plsc.get_sparse_core_info()
    nc, ns = info.num_cores, info.num_subcores
    nw = nc * ns
    epw = ED // nw
    nbt = B // BLANE
    assert ED % nw == 0 and B % BLANE == 0
    mesh = plsc.VectorSubcoreMesh(core_axis_name="c", subcore_axis_name="s")

    @functools.partial(
        pl.kernel,
        mesh=mesh,
        compiler_params=pltpu.CompilerParams(needs_layout_passes=False),
        out_type=jax.ShapeDtypeStruct((ED, D, WD, nbt, H, BLANE), jnp.float32),
        scratch_types=[
            pltpu.VMEM((epw * WPAD,), jnp.float32),
            pltpu.VMEM((B,), jnp.int32),
            pltpu.VMEM((ED,), jnp.float32),
            pltpu.VMEM((nbt, H, BLANE), jnp.float32),
            pltpu.VMEM((nbt, H, BLANE), jnp.float32),
            pltpu.SemaphoreType.DMA,
            pltpu.SemaphoreType.DMA,
        ],
    )
    def lookup(w_hbm, idx_hbm, b_hbm, out_hbm,
               w_v, idx_v, b_v, blk0, blk1, sem0, sem1):
        wid = lax.axis_index("s") * nc + lax.axis_index("c")
        e0 = wid * epw

        pltpu.sync_copy(w_hbm.at[pl.ds(e0 * WPAD, epw * WPAD)], w_v)
        pltpu.sync_copy(idx_hbm, idx_v)
        pltpu.sync_copy(b_hbm, b_v)

        def build_block(le, blk):
            rowbase = le * WPAD
            bias = plsc.load_gather(b_v, [jnp.full((L,), e0 + le, jnp.int32)])

            def sbody(c, acc):
                return acc + w_v[pl.ds(rowbase + L * c, L)]
            acc = lax.fori_loop(
                0, WPAD // L, sbody, jnp.zeros((L,), jnp.float32))
            total = jnp.sum(acc)
            w_e0 = plsc.load_gather(w_v, [jnp.full((L,), rowbase, jnp.int32)])
            mh_vec = (jnp.full((L,), total, jnp.float32) - w_e0) + bias

            def cbody(c, carry):
                cvec = idx_v[pl.ds(L * c, L)]
                g = plsc.load_gather(w_v, [cvec + rowbase])
                val = jnp.where(cvec < NCOND, g + bias, mh_vec)
                k = c // (BLANE // L)
                lanepos = L * (c % (BLANE // L))
                for h in range(H):
                    blk[k, h, pl.ds(lanepos, L)] = val
                return carry
            lax.fori_loop(0, B // L, cbody, 0)

        def fire(le, blk, sem):
            for d in range(D):
                for w in range(WD):
                    pltpu.make_async_copy(
                        blk, out_hbm.at[e0 + le, d, w], sem).start()

        def drain(le, blk, sem):
            for d in range(D):
                for w in range(WD):
                    pltpu.make_async_copy(
                        blk, out_hbm.at[e0 + le, d, w], sem).wait()

        build_block(0, blk0)
        fire(0, blk0, sem0)
        build_block(1, blk1)
        fire(1, blk1, sem1)
        drain(0, blk0, sem0)
        drain(1, blk1, sem1)

    return lookup


def kernel(condition, spatial_shape, W, b):
    dims = jnp.asarray(spatial_shape)
    one = (dims[0] - D + dims[1] - WD + dims[2] - H + 1).astype(jnp.float32)
    B = condition.shape[0]
    w_pad = jnp.pad(W.astype(jnp.float32) * one, ((0, 0), (0, WPAD - NCOND)))
    b_eff = b.astype(jnp.float32) * one
    idx = condition.astype(jnp.int32)
    out6 = _make_lookup(B)(w_pad.reshape(ED * WPAD), idx, b_eff)
    out5 = out6.transpose(3, 5, 0, 1, 2, 4).reshape(B, ED, D, WD, H)
    return out5
